# baseline (speedup 1.0000x reference)
import functools

import jax
import jax.numpy as jnp
from jax import lax
from jax.experimental import pallas as pl
from jax.experimental.pallas import tpu as pltpu
from jax.experimental.pallas import tpu_sc as plsc

BATCH = 16384
MF_DIM = 64
MLP_IN_HALF = 128
H0 = 128
H1 = 64

NC, NS = 2, 16
NW = NC * NS
B_PER_W = BATCH // NW


def _gather_all(gmf_user_p, gmf_item_p, mlp_user, mlp_item,
                gmf_uidx, gmf_iidx, user_idxs, item_idxs):
    mesh = plsc.VectorSubcoreMesh(core_axis_name="c", subcore_axis_name="s")
    row_t = jax.ShapeDtypeStruct((BATCH, 128), jnp.float32)

    @functools.partial(
        pl.kernel,
        mesh=mesh,
        out_type=[row_t, row_t, row_t, row_t],
        scratch_types=[
            pltpu.VMEM((B_PER_W,), jnp.int32),
            pltpu.VMEM((B_PER_W,), jnp.int32),
            pltpu.VMEM((B_PER_W,), jnp.int32),
            pltpu.VMEM((B_PER_W,), jnp.int32),
            pltpu.VMEM((B_PER_W, 128), jnp.float32),
            pltpu.SemaphoreType.DMA,
        ],
    )
    def k(gu_hbm, gi_hbm, mu_hbm, mi_hbm,
          guidx_hbm, giidx_hbm, uidx_hbm, iidx_hbm,
          out_gu, out_gi, out_mu, out_mi,
          guidx_v, giidx_v, uidx_v, iidx_v, rows_v, sem):
        wid = lax.axis_index("s") * NC + lax.axis_index("c")
        base = wid * B_PER_W
        sl = pl.ds(base, B_PER_W)
        pltpu.sync_copy(guidx_hbm.at[sl], guidx_v)
        pltpu.sync_copy(giidx_hbm.at[sl], giidx_v)
        pltpu.sync_copy(uidx_hbm.at[sl], uidx_v)
        pltpu.sync_copy(iidx_hbm.at[sl], iidx_v)

        pltpu.async_copy(gu_hbm.at[guidx_v], rows_v, sem).wait()
        pltpu.sync_copy(rows_v, out_gu.at[sl])
        pltpu.async_copy(gi_hbm.at[giidx_v], rows_v, sem).wait()
        pltpu.sync_copy(rows_v, out_gi.at[sl])
        pltpu.async_copy(mu_hbm.at[uidx_v], rows_v, sem).wait()
        pltpu.sync_copy(rows_v, out_mu.at[sl])
        pltpu.async_copy(mi_hbm.at[iidx_v], rows_v, sem).wait()
        pltpu.sync_copy(rows_v, out_mi.at[sl])

    return k(gmf_user_p, gmf_item_p, mlp_user, mlp_item,
             gmf_uidx, gmf_iidx, user_idxs, item_idxs)


BT = 2048


def _head_body(gup_ref, gip_ref, mu_ref, mi_ref, pu_ref, pi_ref,
               w0u_ref, w0i_ref, b0_ref, w1_ref, b1_ref,
               wfg_ref, wfm_ref, bf_ref, out_ref):
    h0 = jnp.dot(mu_ref[...], w0u_ref[...], preferred_element_type=jnp.float32)
    h0 += jnp.dot(mi_ref[...], w0i_ref[...], preferred_element_type=jnp.float32)
    h0 = jnp.maximum(h0 + b0_ref[...], 0.0)
    h1 = jnp.dot(h0, w1_ref[...], preferred_element_type=jnp.float32)
    h1 = jnp.maximum(h1 + b1_ref[...], 0.0)
    pu = pu_ref[...]
    pi = pi_ref[...]
    gup = gup_ref[...]
    gip = gip_ref[...]
    gu = gup[:, :MF_DIM] + pu * (gup[:, MF_DIM:] - gup[:, :MF_DIM])
    gi = gip[:, :MF_DIM] + pi * (gip[:, MF_DIM:] - gip[:, :MF_DIM])
    gmf = gu * gi
    logit = jnp.sum(gmf * wfg_ref[...], axis=1) + jnp.sum(h1 * wfm_ref[...], axis=1)
    out_ref[...] = jax.nn.sigmoid(logit + bf_ref[0])


def _dense_head(gup, gip, mu, mi, pu, pi, W0, b0, W1, b1, Wf, bf):
    w0u = W0[:, :MLP_IN_HALF].T
    w0i = W0[:, MLP_IN_HALF:].T
    w1 = W1.T
    wfg = Wf[0, :MF_DIM].reshape(1, MF_DIM)
    wfm = Wf[0, MF_DIM:].reshape(1, H1)
    b0r = b0.reshape(1, H0)
    b1r = b1.reshape(1, H1)

    grid = (BATCH // BT,)
    full = lambda shape: pl.BlockSpec(shape, lambda i: (0,) * len(shape))
    return pl.pallas_call(
        _head_body,
        grid=grid,
        in_specs=[
            pl.BlockSpec((BT, 128), lambda i: (i, 0)),
            pl.BlockSpec((BT, 128), lambda i: (i, 0)),
            pl.BlockSpec((BT, MLP_IN_HALF), lambda i: (i, 0)),
            pl.BlockSpec((BT, MLP_IN_HALF), lambda i: (i, 0)),
            pl.BlockSpec((BT, 1), lambda i: (i, 0)),
            pl.BlockSpec((BT, 1), lambda i: (i, 0)),
            full((MLP_IN_HALF, H0)),
            full((MLP_IN_HALF, H0)),
            full((1, H0)),
            full((H0, H1)),
            full((1, H1)),
            full((1, MF_DIM)),
            full((1, H1)),
            full((1,)),
        ],
        out_specs=pl.BlockSpec((BT,), lambda i: (i,)),
        out_shape=jax.ShapeDtypeStruct((BATCH,), jnp.float32),
    )(gup, gip, mu, mi, pu, pi, w0u, w0i, b0r, w1, b1r, wfg, wfm, bf)


def kernel(user_idxs, item_idxs, gmf_user, gmf_item, mlp_user, mlp_item,
           W0, b0, W1, b1, Wf, bf):
    n_users = gmf_user.shape[0]
    n_items = gmf_item.shape[0]
    gmf_user_p = gmf_user.reshape(n_users // 2, 2 * MF_DIM)
    gmf_item_p = gmf_item.reshape(n_items // 2, 2 * MF_DIM)
    gmf_uidx = lax.shift_right_logical(user_idxs, 1)
    gmf_iidx = lax.shift_right_logical(item_idxs, 1)
    pu = jnp.bitwise_and(user_idxs, 1).astype(jnp.float32).reshape(BATCH, 1)
    pi = jnp.bitwise_and(item_idxs, 1).astype(jnp.float32).reshape(BATCH, 1)

    gup, gip, mu, mi = _gather_all(gmf_user_p, gmf_item_p, mlp_user, mlp_item,
                                   gmf_uidx, gmf_iidx, user_idxs, item_idxs)
    return _dense_head(gup, gip, mu, mi, pu, pi, W0, b0, W1, b1, Wf, bf)
